# Initial kernel scaffold; baseline (speedup 1.0000x reference)
import jax
import jax.numpy as jnp
from jax.experimental import pallas as pl


def kernel(feat_emb, nodes_batch, labels_batch, anomaly_list, norm_list, W1, b1, W2, b2):
    raise NotImplementedError("write your pallas kernel here")


# speedup vs baseline: 9.8766x; 9.8766x over previous
import jax
import jax.numpy as jnp
from jax.experimental import pallas as pl

N, D, A, NNORM, NB = 100000, 128, 64, 1024, 1024
HIDDEN_K, EPISODES, STEP_SIZE, INI_TH = 16, 2, 0.02, 0.5

R = 2048
NPAD = ((N + R - 1) // R) * R
NBLK = NPAD // R
EPS = 1e-8
NEG = -jnp.inf


def _main_body(feat, w1t, b1, w2t, b2, raw_ano, ano_idx, pred_o, sims_o):
    i = pl.program_id(0)
    x = feat[...]
    h = jnp.maximum(
        jnp.dot(x, w1t[...], preferred_element_type=jnp.float32) + b1[...], 0.0)
    pred_o[...] = jnp.dot(h, w2t[...], preferred_element_type=jnp.float32) + b2[0, 0]

    nrm = jnp.sqrt(jnp.sum(x * x, axis=1, keepdims=True))
    nf = x / jnp.maximum(nrm, EPS)

    a = raw_ano[...]
    anrm = jnp.sqrt(jnp.sum(a * a, axis=1, keepdims=True))
    an = a / jnp.maximum(anrm, EPS)

    sim = jax.lax.dot_general(an, nf, (((1,), (1,)), ((), ())),
                              preferred_element_type=jnp.float32)
    col = i * R + jax.lax.broadcasted_iota(jnp.int32, (1, R), 1)
    sim = jnp.where(col == ano_idx[...], NEG, sim)
    sim = jnp.where(col >= N, NEG, sim)
    sims_o[...] = sim


def kernel(feat_emb, nodes_batch, labels_batch, anomaly_list, norm_list, W1, b1, W2, b2):
    k = HIDDEN_K
    feat_pad = jnp.pad(feat_emb, ((0, NPAD - N), (0, 0)))
    w1t = W1.T
    w2t = W2.T
    b1_2d = b1.reshape(1, 8 * D)
    b2_2d = b2.reshape(1, 1)
    raw_ano = feat_emb[anomaly_list]
    ano_idx = anomaly_list.astype(jnp.int32).reshape(A, 1)

    pred_pad, sims = pl.pallas_call(
        _main_body,
        grid=(NBLK,),
        in_specs=[
            pl.BlockSpec((R, D), lambda i: (i, 0)),
            pl.BlockSpec((D, 8 * D), lambda i: (0, 0)),
            pl.BlockSpec((1, 8 * D), lambda i: (0, 0)),
            pl.BlockSpec((8 * D, 1), lambda i: (0, 0)),
            pl.BlockSpec((1, 1), lambda i: (0, 0)),
            pl.BlockSpec((A, D), lambda i: (0, 0)),
            pl.BlockSpec((A, 1), lambda i: (0, 0)),
        ],
        out_specs=[
            pl.BlockSpec((R, 1), lambda i: (i, 0)),
            pl.BlockSpec((A, R), lambda i: (0, i)),
        ],
        out_shape=[
            jax.ShapeDtypeStruct((NPAD, 1), jnp.float32),
            jax.ShapeDtypeStruct((A, NPAD), jnp.float32),
        ],
    )(feat_pad, w1t, b1_2d, w2t, b2_2d, raw_ano, ano_idx)
    pred_score = pred_pad[:N]

    sims_near, nearest = jax.lax.top_k(sims, k)

    norms = jnp.linalg.norm(feat_emb, axis=1, keepdims=True)
    nf = feat_emb / jnp.maximum(norms, EPS)
    nf_ano = nf[anomaly_list]
    nf_no = nf[norm_list]
    std_ab = (nf_ano @ nf_ano.T).mean(axis=1)
    std_no = (nf_ano @ nf_no.T).mean(axis=1)
    nf_near = nf[nearest]
    nei_ab = (nf_near @ nf_ano.T).mean(axis=2)
    nei_no = (nf_near @ nf_no.T).mean(axis=2)

    th = jnp.full((A,), INI_TH, dtype=jnp.float32)
    rewards_log = []
    for _ in range(EPISODES):
        cond = sims_near >= th[:, None]
        good = jnp.where(cond, nei_ab >= std_ab[:, None], nei_no <= std_no[:, None])
        reward = good.sum(axis=1).astype(jnp.float32)
        th = jnp.where(reward >= 0.5 * k, th - STEP_SIZE, th + STEP_SIZE)
        rewards_log.append(reward)
    rewards = jnp.stack(rewards_log, axis=1)

    new_th = jnp.where(th >= 1.0, 0.999, th)
    new_th = jnp.where(new_th <= 0.0, 0.001, new_th)
    mask = sims_near > new_th[:, None]

    return (nearest, sims_near, mask, rewards, new_th, pred_score)


# speedup vs baseline: 14.4629x; 1.4644x over previous
import functools

import jax
import jax.numpy as jnp
from jax import lax
from jax.experimental import pallas as pl
from jax.experimental.pallas import tpu as pltpu
from jax.experimental.pallas import tpu_sc as plsc

N, D, A, NNORM, NB = 100000, 128, 64, 1024, 1024
HIDDEN_K, EPISODES, STEP_SIZE, INI_TH = 16, 2, 0.02, 0.5

R = 2048
NPAD = ((N + R - 1) // R) * R
NBLK = NPAD // R
CHUNK = 128
NCH = NPAD // CHUNK
EPS = 1e-8
NEG = -jnp.inf

NC, NS, L = 2, 16, 16
NW = NC * NS
GB = 1280
GPW = GB // NW
ROWS_PW = A // NW

_SC_MESH = dict(core_axis_name="c", subcore_axis_name="s")


def _g1_body(feat_hbm, idx_hbm, out_hbm, idx_v, rows_v, sem):
    wid = lax.axis_index("s") * NC + lax.axis_index("c")
    base = wid * GPW
    pltpu.sync_copy(idx_hbm.at[pl.ds(base, GPW)], idx_v)
    gcp = pltpu.make_async_copy(feat_hbm.at[idx_v], rows_v, sem)
    gcp.start()
    gcp.wait()
    pltpu.sync_copy(rows_v, out_hbm.at[pl.ds(base, GPW)])


def _gather_rows(feat_emb, idx_pad):
    return pl.kernel(
        _g1_body,
        out_type=jax.ShapeDtypeStruct((GB, D), jnp.float32),
        compiler_params=pltpu.CompilerParams(needs_layout_passes=False),
        mesh=plsc.VectorSubcoreMesh(**_SC_MESH),
        scratch_types=[
            pltpu.VMEM((GPW,), jnp.int32),
            pltpu.VMEM((GPW, D), jnp.float32),
            pltpu.SemaphoreType.DMA,
        ],
    )(feat_emb, idx_pad)


def _main_body(feat, w1t, b1, w2t, b2, raw_ano, ano_idx, pred_o, sims_o, m_o):
    i = pl.program_id(0)
    x = feat[...]
    h = jnp.maximum(
        jnp.dot(x, w1t[...], preferred_element_type=jnp.float32) + b1[...], 0.0)
    pred_o[...] = jnp.dot(h, w2t[...], preferred_element_type=jnp.float32) + b2[0, 0]

    nrm = jnp.sqrt(jnp.sum(x * x, axis=1, keepdims=True))
    nf = x / jnp.maximum(nrm, EPS)

    a = raw_ano[...]
    anrm = jnp.sqrt(jnp.sum(a * a, axis=1, keepdims=True))
    an = a / jnp.maximum(anrm, EPS)

    sim = lax.dot_general(an, nf, (((1,), (1,)), ((), ())),
                          preferred_element_type=jnp.float32)
    col = i * R + lax.broadcasted_iota(jnp.int32, (1, R), 1)
    sim = jnp.where(col == ano_idx[...], NEG, sim)
    sim = jnp.where(col >= N, NEG, sim)
    sims_o[...] = sim
    m_o[...] = jnp.max(sim.reshape(A, R // CHUNK, CHUNK), axis=2).reshape(
        1, A, R // CHUNK)


def _main_pass(feat_pad, w1t, b1_2d, w2t, b2_2d, raw_ano, ano_idx):
    return pl.pallas_call(
        _main_body,
        grid=(NBLK,),
        in_specs=[
            pl.BlockSpec((R, D), lambda i: (i, 0)),
            pl.BlockSpec((D, 8 * D), lambda i: (0, 0)),
            pl.BlockSpec((1, 8 * D), lambda i: (0, 0)),
            pl.BlockSpec((8 * D, 1), lambda i: (0, 0)),
            pl.BlockSpec((1, 1), lambda i: (0, 0)),
            pl.BlockSpec((A, D), lambda i: (0, 0)),
            pl.BlockSpec((A, 1), lambda i: (0, 0)),
        ],
        out_specs=[
            pl.BlockSpec((R, 1), lambda i: (i, 0)),
            pl.BlockSpec((A, R), lambda i: (0, i)),
            pl.BlockSpec((1, A, R // CHUNK), lambda i: (i, 0, 0)),
        ],
        out_shape=[
            jax.ShapeDtypeStruct((NPAD, 1), jnp.float32),
            jax.ShapeDtypeStruct((A, NPAD), jnp.float32),
            jax.ShapeDtypeStruct((NBLK, A, R // CHUNK), jnp.float32),
        ],
    )(feat_pad, w1t, b1_2d, w2t, b2_2d, raw_ano, ano_idx)


def _merge16(vals, idxs, x, xi):
    xs, xis = plsc.sort_key_val(x, xi, descending=True)
    rxs = lax.rev(xs, (0,))
    rxis = lax.rev(xis, (0,))
    take_a = (vals > rxs) | ((vals == rxs) & (idxs < rxis))
    mv = jnp.where(take_a, vals, rxs)
    mi = jnp.where(take_a, idxs, rxis)
    nv, ni = plsc.sort_key_val(mv, mi, descending=True)
    return nv, ni


def _topk_body(sims_hbm, m_hbm, feat_hbm, vals_out, idx_out, near_out,
               buf, mbuf, vout, iout, rows_v, sem):
    wid = lax.axis_index("s") * NC + lax.axis_index("c")
    iota16 = lax.iota(jnp.int32, L)
    minf = jnp.full((L,), NEG, dtype=jnp.float32)

    for rr in range(ROWS_PW):
        row = wid * ROWS_PW + rr
        cp = pltpu.make_async_copy(sims_hbm.at[row], buf, sem)
        cp.start()
        pltpu.sync_copy(m_hbm.at[row], mbuf)

        def p1_body(g, vals):
            cm = mbuf[pl.ds(g * L, L)]
            thr = jnp.min(vals)

            def do_merge(v):
                rxs = lax.rev(plsc.sort_key_val(cm, iota16, descending=True)[0], (0,))
                return plsc.sort_key_val(jnp.maximum(v, rxs), iota16,
                                         descending=True)[0]

            return lax.cond(jnp.any(cm > thr), do_merge, lambda v: v, vals)

        t0 = jnp.min(lax.fori_loop(0, NCH // L, p1_body, minf))
        cp.wait()

        def p2_body(g, carry):
            vals, idxs = carry
            cm = mbuf[pl.ds(g * L, L)]

            def scan_group(carry):
                base = g * L * CHUNK

                def vbody(j, c):
                    off = base + j * L
                    x = buf[pl.ds(off, L)]
                    thr = jnp.min(c[0])
                    xi = off + iota16
                    return lax.cond(
                        jnp.any(x > thr),
                        lambda c: _merge16(c[0], c[1], x, xi),
                        lambda c: c, c)

                return lax.fori_loop(0, CHUNK, vbody, carry)

            return lax.cond(jnp.any(cm >= t0), scan_group,
                            lambda c: c, (vals, idxs))

        vals, idxs = lax.fori_loop(0, NCH // L, p2_body,
                                   (minf, jnp.zeros((L,), jnp.int32)))
        vout[...] = vals
        iout[...] = idxs
        pltpu.sync_copy(vout, vals_out.at[row])
        pltpu.sync_copy(iout, idx_out.at[row])
        ncp = pltpu.make_async_copy(feat_hbm.at[iout], rows_v, sem)
        ncp.start()
        ncp.wait()
        pltpu.sync_copy(rows_v, near_out.at[pl.ds(row * HIDDEN_K, HIDDEN_K)])


def _topk_pass(sims, m_arr, feat_emb):
    return pl.kernel(
        _topk_body,
        out_type=[
            jax.ShapeDtypeStruct((A, HIDDEN_K), jnp.float32),
            jax.ShapeDtypeStruct((A, HIDDEN_K), jnp.int32),
            jax.ShapeDtypeStruct((A * HIDDEN_K, D), jnp.float32),
        ],
        compiler_params=pltpu.CompilerParams(needs_layout_passes=False),
        mesh=plsc.VectorSubcoreMesh(**_SC_MESH),
        scratch_types=[
            pltpu.VMEM((NPAD,), jnp.float32),
            pltpu.VMEM((NCH,), jnp.float32),
            pltpu.VMEM((HIDDEN_K,), jnp.float32),
            pltpu.VMEM((HIDDEN_K,), jnp.int32),
            pltpu.VMEM((HIDDEN_K, D), jnp.float32),
            pltpu.SemaphoreType.DMA,
        ],
    )(sims, m_arr, feat_emb)


def _stats_body(raw_ano, raw_no, raw_near, std_ab_o, std_no_o, nei_ab_o, nei_no_o):
    a = raw_ano[...]
    an = a / jnp.maximum(jnp.sqrt(jnp.sum(a * a, axis=1, keepdims=True)), EPS)
    no = raw_no[...]
    non = no / jnp.maximum(jnp.sqrt(jnp.sum(no * no, axis=1, keepdims=True)), EPS)
    nr = raw_near[...]
    nrn = nr / jnp.maximum(jnp.sqrt(jnp.sum(nr * nr, axis=1, keepdims=True)), EPS)

    dn = (((1,), (1,)), ((), ()))
    std_ab_o[...] = jnp.mean(
        lax.dot_general(an, an, dn, preferred_element_type=jnp.float32),
        axis=1, keepdims=True)
    std_no_o[...] = jnp.mean(
        lax.dot_general(an, non, dn, preferred_element_type=jnp.float32),
        axis=1, keepdims=True)
    nei_ab_o[...] = jnp.mean(
        lax.dot_general(nrn, an, dn, preferred_element_type=jnp.float32),
        axis=1, keepdims=True)
    nei_no_o[...] = jnp.mean(
        lax.dot_general(nrn, non, dn, preferred_element_type=jnp.float32),
        axis=1, keepdims=True)


def _stats_pass(raw_ano, raw_no, raw_near):
    return pl.pallas_call(
        _stats_body,
        out_shape=[
            jax.ShapeDtypeStruct((A, 1), jnp.float32),
            jax.ShapeDtypeStruct((A, 1), jnp.float32),
            jax.ShapeDtypeStruct((A * HIDDEN_K, 1), jnp.float32),
            jax.ShapeDtypeStruct((A * HIDDEN_K, 1), jnp.float32),
        ],
    )(raw_ano, raw_no, raw_near)


def _rl_body(sims_near, std_ab, std_no, nei_ab, nei_no,
             rewards_o, newth_o, mask_o):
    sims = sims_near[...]
    sab = std_ab[...]
    sno = std_no[...]
    nab = nei_ab[...]
    nno = nei_no[...]
    th = jnp.full((A, 1), INI_TH, dtype=jnp.float32)
    rws = []
    for _ in range(EPISODES):
        cond = sims >= th
        good = jnp.where(cond, (nab >= sab).astype(jnp.float32),
                         (nno <= sno).astype(jnp.float32))
        reward = jnp.sum(good, axis=1, keepdims=True)
        th = jnp.where(reward >= 0.5 * HIDDEN_K, th - STEP_SIZE, th + STEP_SIZE)
        rws.append(reward)
    rewards_o[...] = jnp.concatenate(rws, axis=1)
    new_th = jnp.where(th >= 1.0, 0.999, th)
    new_th = jnp.where(new_th <= 0.0, 0.001, new_th)
    newth_o[...] = new_th
    mask_o[...] = (sims > new_th).astype(jnp.int32)


def _rl_pass(sims_near, std_ab, std_no, nei_ab, nei_no):
    return pl.pallas_call(
        _rl_body,
        out_shape=[
            jax.ShapeDtypeStruct((A, EPISODES), jnp.float32),
            jax.ShapeDtypeStruct((A, 1), jnp.float32),
            jax.ShapeDtypeStruct((A, HIDDEN_K), jnp.int32),
        ],
    )(sims_near, std_ab, std_no, nei_ab, nei_no)


def kernel(feat_emb, nodes_batch, labels_batch, anomaly_list, norm_list, W1, b1, W2, b2):
    feat_pad = jnp.pad(feat_emb, ((0, NPAD - N), (0, 0)))
    w1t = W1.T
    w2t = W2.T
    b1_2d = b1.reshape(1, 8 * D)
    b2_2d = b2.reshape(1, 1)

    idx_pad = jnp.concatenate([
        anomaly_list.astype(jnp.int32), norm_list.astype(jnp.int32),
        jnp.zeros((GB - A - NNORM,), jnp.int32)])
    gath = _gather_rows(feat_emb, idx_pad)
    raw_ano = gath[:A]
    raw_no = gath[A:A + NNORM]

    ano_idx = anomaly_list.astype(jnp.int32).reshape(A, 1)
    pred_pad, sims, m3 = _main_pass(feat_pad, w1t, b1_2d, w2t, b2_2d,
                                    raw_ano, ano_idx)
    pred_score = pred_pad[:N]
    m_arr = m3.transpose(1, 0, 2).reshape(A, NCH)

    _TOPK_SC = True
    if _TOPK_SC:
        sims_near, nearest, raw_near = _topk_pass(sims, m_arr, feat_emb)
    else:
        sims_near, nearest = lax.top_k(sims, HIDDEN_K)
        raw_near = feat_emb[nearest.reshape(-1)]

    std_ab, std_no, nei_ab_f, nei_no_f = _stats_pass(raw_ano, raw_no, raw_near)
    rewards, newth, mask_i = _rl_pass(
        sims_near, std_ab, std_no,
        nei_ab_f.reshape(A, HIDDEN_K), nei_no_f.reshape(A, HIDDEN_K))

    return (nearest, sims_near, mask_i.astype(jnp.bool_), rewards,
            newth.reshape(A), pred_score)


# speedup vs baseline: 21.3290x; 1.4747x over previous
import functools

import jax
import jax.numpy as jnp
from jax import lax
from jax.experimental import pallas as pl
from jax.experimental.pallas import tpu as pltpu
from jax.experimental.pallas import tpu_sc as plsc

N, D, A, NNORM, NB = 100000, 128, 64, 1024, 1024
HIDDEN_K, EPISODES, STEP_SIZE, INI_TH = 16, 2, 0.02, 0.5

R = 2048
NPAD = ((N + R - 1) // R) * R
NBLK = NPAD // R
CHUNK = 128
NCH = NPAD // CHUNK
EPS = 1e-8
NEG = -jnp.inf

NC, NS, L = 2, 16, 16
NW = NC * NS
GB = 1280
GPW = GB // NW
ROWS_PW = A // NW

_SC_MESH = dict(core_axis_name="c", subcore_axis_name="s")


def _g1_body(feat_hbm, idx_hbm, out_hbm, idx_v, rows_v, sem):
    wid = lax.axis_index("s") * NC + lax.axis_index("c")
    base = wid * GPW
    pltpu.sync_copy(idx_hbm.at[pl.ds(base, GPW)], idx_v)
    gcp = pltpu.make_async_copy(feat_hbm.at[idx_v], rows_v, sem)
    gcp.start()
    gcp.wait()
    pltpu.sync_copy(rows_v, out_hbm.at[pl.ds(base, GPW)])


def _gather_rows(feat_emb, idx_pad):
    return pl.kernel(
        _g1_body,
        out_type=jax.ShapeDtypeStruct((GB, D), jnp.float32),
        compiler_params=pltpu.CompilerParams(needs_layout_passes=False),
        mesh=plsc.VectorSubcoreMesh(**_SC_MESH),
        scratch_types=[
            pltpu.VMEM((GPW,), jnp.int32),
            pltpu.VMEM((GPW, D), jnp.float32),
            pltpu.SemaphoreType.DMA,
        ],
    )(feat_emb, idx_pad)


def _main_body(feat, w1t, b1, w2t, b2, raw_ano, ano_idx, pred_o, sims_o, m_o):
    i = pl.program_id(0)
    x = feat[...]
    h = jnp.maximum(
        jnp.dot(x.astype(jnp.bfloat16), w1t[...].astype(jnp.bfloat16),
                preferred_element_type=jnp.float32) + b1[...], 0.0)
    pred_o[...] = jnp.dot(h.astype(jnp.bfloat16), w2t[...].astype(jnp.bfloat16),
                          preferred_element_type=jnp.float32) + b2[0, 0]

    nrm = jnp.sqrt(jnp.sum(x * x, axis=1, keepdims=True))
    nf = x / jnp.maximum(nrm, EPS)

    a = raw_ano[...]
    anrm = jnp.sqrt(jnp.sum(a * a, axis=1, keepdims=True))
    an = a / jnp.maximum(anrm, EPS)

    sim = lax.dot_general(an, nf, (((1,), (1,)), ((), ())),
                          preferred_element_type=jnp.float32)
    col = i * R + lax.broadcasted_iota(jnp.int32, (1, R), 1)
    sim = jnp.where(col == ano_idx[...], NEG, sim)
    sim = jnp.where(col >= N, NEG, sim)
    sims_o[...] = sim
    m_o[...] = jnp.max(sim.reshape(A, R // CHUNK, CHUNK), axis=2).reshape(
        1, A, R // CHUNK)


def _main_pass(feat_pad, w1t, b1_2d, w2t, b2_2d, raw_ano, ano_idx):
    return pl.pallas_call(
        _main_body,
        grid=(NBLK,),
        in_specs=[
            pl.BlockSpec((R, D), lambda i: (i, 0)),
            pl.BlockSpec((D, 8 * D), lambda i: (0, 0)),
            pl.BlockSpec((1, 8 * D), lambda i: (0, 0)),
            pl.BlockSpec((8 * D, 1), lambda i: (0, 0)),
            pl.BlockSpec((1, 1), lambda i: (0, 0)),
            pl.BlockSpec((A, D), lambda i: (0, 0)),
            pl.BlockSpec((A, 1), lambda i: (0, 0)),
        ],
        out_specs=[
            pl.BlockSpec((R, 1), lambda i: (i, 0)),
            pl.BlockSpec((A, R), lambda i: (0, i)),
            pl.BlockSpec((1, A, R // CHUNK), lambda i: (i, 0, 0)),
        ],
        out_shape=[
            jax.ShapeDtypeStruct((NPAD, 1), jnp.float32),
            jax.ShapeDtypeStruct((A, NPAD), jnp.float32),
            jax.ShapeDtypeStruct((NBLK, A, R // CHUNK), jnp.float32),
        ],
    )(feat_pad, w1t, b1_2d, w2t, b2_2d, raw_ano, ano_idx)


def _merge16(vals, idxs, x, xi):
    xs, xis = plsc.sort_key_val(x, xi, descending=True)
    rxs = lax.rev(xs, (0,))
    rxis = lax.rev(xis, (0,))
    take_a = (vals > rxs) | ((vals == rxs) & (idxs < rxis))
    mv = jnp.where(take_a, vals, rxs)
    mi = jnp.where(take_a, idxs, rxis)
    nv, ni = plsc.sort_key_val(mv, mi, descending=True)
    return nv, ni


def _topk_body(sims_hbm, m_hbm, feat_hbm, vals_out, idx_out, near_out,
               buf, mbuf, idref, vout, iout, rows_v, sem):
    wid = lax.axis_index("s") * NC + lax.axis_index("c")
    iota16 = lax.iota(jnp.int32, L)
    minf = jnp.full((L,), NEG, dtype=jnp.float32)

    for rr in range(ROWS_PW):
        row = wid * ROWS_PW + rr
        pltpu.sync_copy(m_hbm.at[row], mbuf)

        def p1_body(g, carry):
            vals, thr = carry
            cm = mbuf[pl.ds(g * L, L)]

            def do_merge(c):
                rxs = lax.rev(plsc.sort_key_val(cm, iota16, descending=True)[0], (0,))
                nv = plsc.sort_key_val(jnp.maximum(c[0], rxs), iota16,
                                       descending=True)[0]
                return nv, jnp.min(nv)

            return lax.cond(jnp.any(cm > thr), do_merge, lambda c: c, carry)

        _, t0 = lax.fori_loop(0, NCH // L, p1_body, (minf, jnp.min(minf)))

        def pa_group(g, slot):
            cm = mbuf[pl.ds(g * L, L)]

            def hit_group(s):
                for lane in range(L):
                    def issue(s2, _lane=lane):
                        chunk = g * L + _lane
                        idref[s2] = chunk
                        pltpu.make_async_copy(
                            sims_hbm.at[row, pl.ds(chunk * CHUNK, CHUNK)],
                            buf.at[pl.ds(s2 * CHUNK, CHUNK)], sem).start()
                        return s2 + 1

                    s = lax.cond(cm[lane] >= t0, issue, lambda s2: s2, s)
                return s

            return lax.cond(jnp.any(cm >= t0), hit_group, lambda s: s, slot)

        slot = lax.fori_loop(0, NCH // L, pa_group, jnp.int32(0))

        def pb(j, z):
            pltpu.make_async_copy(sims_hbm.at[row, pl.ds(0, CHUNK)],
                                  buf.at[pl.ds(0, CHUNK)], sem).wait()
            return z

        lax.fori_loop(0, slot, pb, jnp.int32(0))

        def pc(j, carry):
            chunk = idref[j]
            cb = j * CHUNK
            for v in range(CHUNK // L):
                off = cb + v * L
                x = buf[pl.ds(off, L)]
                xi = chunk * CHUNK + v * L + iota16

                def mg(c, x=x, xi=xi):
                    nv, ni = _merge16(c[0], c[1], x, xi)
                    return nv, ni, jnp.min(nv)

                carry = lax.cond(jnp.any(x > carry[2]), mg,
                                 lambda c: c, carry)
            return carry

        vals, idxs, _ = lax.fori_loop(
            0, slot, pc,
            (minf, jnp.zeros((L,), jnp.int32), jnp.min(minf)))
        vout[...] = vals
        iout[...] = idxs
        pltpu.sync_copy(vout, vals_out.at[row])
        pltpu.sync_copy(iout, idx_out.at[row])
        ncp = pltpu.make_async_copy(feat_hbm.at[iout], rows_v, sem)
        ncp.start()
        ncp.wait()
        pltpu.sync_copy(rows_v, near_out.at[pl.ds(row * HIDDEN_K, HIDDEN_K)])


def _topk_pass(sims, m_arr, feat_emb):
    return pl.kernel(
        _topk_body,
        out_type=[
            jax.ShapeDtypeStruct((A, HIDDEN_K), jnp.float32),
            jax.ShapeDtypeStruct((A, HIDDEN_K), jnp.int32),
            jax.ShapeDtypeStruct((A * HIDDEN_K, D), jnp.float32),
        ],
        compiler_params=pltpu.CompilerParams(needs_layout_passes=False),
        mesh=plsc.VectorSubcoreMesh(**_SC_MESH),
        scratch_types=[
            pltpu.VMEM((NPAD,), jnp.float32),
            pltpu.VMEM((NCH,), jnp.float32),
            pltpu.SMEM((NCH,), jnp.int32),
            pltpu.VMEM((HIDDEN_K,), jnp.float32),
            pltpu.VMEM((HIDDEN_K,), jnp.int32),
            pltpu.VMEM((HIDDEN_K, D), jnp.float32),
            pltpu.SemaphoreType.DMA,
        ],
    )(sims, m_arr, feat_emb)


def _stats_body(raw_ano, raw_no, raw_near, std_ab_o, std_no_o, nei_ab_o, nei_no_o):
    a = raw_ano[...]
    an = a / jnp.maximum(jnp.sqrt(jnp.sum(a * a, axis=1, keepdims=True)), EPS)
    no = raw_no[...]
    non = no / jnp.maximum(jnp.sqrt(jnp.sum(no * no, axis=1, keepdims=True)), EPS)
    nr = raw_near[...]
    nrn = nr / jnp.maximum(jnp.sqrt(jnp.sum(nr * nr, axis=1, keepdims=True)), EPS)

    dn = (((1,), (1,)), ((), ()))
    std_ab_o[...] = jnp.mean(
        lax.dot_general(an, an, dn, preferred_element_type=jnp.float32),
        axis=1, keepdims=True)
    std_no_o[...] = jnp.mean(
        lax.dot_general(an, non, dn, preferred_element_type=jnp.float32),
        axis=1, keepdims=True)
    nei_ab_o[...] = jnp.mean(
        lax.dot_general(nrn, an, dn, preferred_element_type=jnp.float32),
        axis=1, keepdims=True)
    nei_no_o[...] = jnp.mean(
        lax.dot_general(nrn, non, dn, preferred_element_type=jnp.float32),
        axis=1, keepdims=True)


def _stats_pass(raw_ano, raw_no, raw_near):
    return pl.pallas_call(
        _stats_body,
        out_shape=[
            jax.ShapeDtypeStruct((A, 1), jnp.float32),
            jax.ShapeDtypeStruct((A, 1), jnp.float32),
            jax.ShapeDtypeStruct((A * HIDDEN_K, 1), jnp.float32),
            jax.ShapeDtypeStruct((A * HIDDEN_K, 1), jnp.float32),
        ],
    )(raw_ano, raw_no, raw_near)


def _rl_body(sims_near, std_ab, std_no, nei_ab, nei_no,
             rewards_o, newth_o, mask_o):
    sims = sims_near[...]
    sab = std_ab[...]
    sno = std_no[...]
    nab = nei_ab[...]
    nno = nei_no[...]
    th = jnp.full((A, 1), INI_TH, dtype=jnp.float32)
    rws = []
    for _ in range(EPISODES):
        cond = sims >= th
        good = jnp.where(cond, (nab >= sab).astype(jnp.float32),
                         (nno <= sno).astype(jnp.float32))
        reward = jnp.sum(good, axis=1, keepdims=True)
        th = jnp.where(reward >= 0.5 * HIDDEN_K, th - STEP_SIZE, th + STEP_SIZE)
        rws.append(reward)
    rewards_o[...] = jnp.concatenate(rws, axis=1)
    new_th = jnp.where(th >= 1.0, 0.999, th)
    new_th = jnp.where(new_th <= 0.0, 0.001, new_th)
    newth_o[...] = new_th
    mask_o[...] = (sims > new_th).astype(jnp.int32)


def _rl_pass(sims_near, std_ab, std_no, nei_ab, nei_no):
    return pl.pallas_call(
        _rl_body,
        out_shape=[
            jax.ShapeDtypeStruct((A, EPISODES), jnp.float32),
            jax.ShapeDtypeStruct((A, 1), jnp.float32),
            jax.ShapeDtypeStruct((A, HIDDEN_K), jnp.int32),
        ],
    )(sims_near, std_ab, std_no, nei_ab, nei_no)


def kernel(feat_emb, nodes_batch, labels_batch, anomaly_list, norm_list, W1, b1, W2, b2):
    feat_pad = jnp.pad(feat_emb, ((0, NPAD - N), (0, 0)))
    w1t = W1.T
    w2t = W2.T
    b1_2d = b1.reshape(1, 8 * D)
    b2_2d = b2.reshape(1, 1)

    idx_pad = jnp.concatenate([
        anomaly_list.astype(jnp.int32), norm_list.astype(jnp.int32),
        jnp.zeros((GB - A - NNORM,), jnp.int32)])
    gath = _gather_rows(feat_emb, idx_pad)
    raw_ano = gath[:A]
    raw_no = gath[A:A + NNORM]

    ano_idx = anomaly_list.astype(jnp.int32).reshape(A, 1)
    pred_pad, sims, m3 = _main_pass(feat_pad, w1t, b1_2d, w2t, b2_2d,
                                    raw_ano, ano_idx)
    pred_score = pred_pad[:N]
    m_arr = m3.transpose(1, 0, 2).reshape(A, NCH)

    _TOPK_SC = True
    if _TOPK_SC:
        sims_near, nearest, raw_near = _topk_pass(sims, m_arr, feat_emb)
    else:
        sims_near, nearest = lax.top_k(sims, HIDDEN_K)
        raw_near = feat_emb[nearest.reshape(-1)]

    std_ab, std_no, nei_ab_f, nei_no_f = _stats_pass(raw_ano, raw_no, raw_near)
    rewards, newth, mask_i = _rl_pass(
        sims_near, std_ab, std_no,
        nei_ab_f.reshape(A, HIDDEN_K), nei_no_f.reshape(A, HIDDEN_K))

    return (nearest, sims_near, mask_i.astype(jnp.bool_), rewards,
            newth.reshape(A), pred_score)


# speedup vs baseline: 25.3363x; 1.1879x over previous
import functools

import jax
import jax.numpy as jnp
from jax import lax
from jax.experimental import pallas as pl
from jax.experimental.pallas import tpu as pltpu
from jax.experimental.pallas import tpu_sc as plsc

N, D, A, NNORM, NB = 100000, 128, 64, 1024, 1024
HIDDEN_K, EPISODES, STEP_SIZE, INI_TH = 16, 2, 0.02, 0.5

R = 4096
NPAD = ((N + R - 1) // R) * R
NBLK = NPAD // R
CHUNK = 128
NCH = NPAD // CHUNK
EPS = 1e-8
NEG = -jnp.inf

NC, NS, L = 2, 16, 16
NW = NC * NS
GB = 1280
GPW = GB // NW
ROWS_PW = A // NW

_SC_MESH = dict(core_axis_name="c", subcore_axis_name="s")


def _g1_body(feat_hbm, idx_hbm, out_hbm, idx_v, rows_v, sem):
    wid = lax.axis_index("s") * NC + lax.axis_index("c")
    base = wid * GPW
    pltpu.sync_copy(idx_hbm.at[pl.ds(base, GPW)], idx_v)
    gcp = pltpu.make_async_copy(feat_hbm.at[idx_v], rows_v, sem)
    gcp.start()
    gcp.wait()
    pltpu.sync_copy(rows_v, out_hbm.at[pl.ds(base, GPW)])


def _gather_rows(feat_emb, idx_pad):
    return pl.kernel(
        _g1_body,
        out_type=jax.ShapeDtypeStruct((GB, D), jnp.float32),
        compiler_params=pltpu.CompilerParams(needs_layout_passes=False),
        mesh=plsc.VectorSubcoreMesh(**_SC_MESH),
        scratch_types=[
            pltpu.VMEM((GPW,), jnp.int32),
            pltpu.VMEM((GPW, D), jnp.float32),
            pltpu.SemaphoreType.DMA,
        ],
    )(feat_emb, idx_pad)


def _main_body(feat, w1t, b1, w2t, b2, raw_ano, ano_idx, pred_o, sims_o, m_o):
    i = pl.program_id(0)
    x = feat[...]
    h = jnp.maximum(
        jnp.dot(x.astype(jnp.bfloat16), w1t[...].astype(jnp.bfloat16),
                preferred_element_type=jnp.float32) + b1[...], 0.0)
    pred_o[...] = jnp.sum(h * w2t[...], axis=1, keepdims=True) + b2[0, 0]

    nrm = jnp.sqrt(jnp.sum(x * x, axis=1, keepdims=True))
    nf = x / jnp.maximum(nrm, EPS)

    a = raw_ano[...]
    anrm = jnp.sqrt(jnp.sum(a * a, axis=1, keepdims=True))
    an = a / jnp.maximum(anrm, EPS)

    sim = lax.dot_general(an, nf, (((1,), (1,)), ((), ())),
                          preferred_element_type=jnp.float32)
    col = i * R + lax.broadcasted_iota(jnp.int32, (1, R), 1)
    sim = jnp.where(col == ano_idx[...], NEG, sim)
    sim = jnp.where(col >= N, NEG, sim)
    sims_o[...] = sim
    m_o[...] = jnp.max(sim.reshape(A, R // CHUNK, CHUNK), axis=2).reshape(
        1, A, R // CHUNK)


def _main_pass(feat_pad, w1t, b1_2d, w2t, b2_2d, raw_ano, ano_idx):
    return pl.pallas_call(
        _main_body,
        grid=(NBLK,),
        in_specs=[
            pl.BlockSpec((R, D), lambda i: (i, 0)),
            pl.BlockSpec((D, 8 * D), lambda i: (0, 0)),
            pl.BlockSpec((1, 8 * D), lambda i: (0, 0)),
            pl.BlockSpec((1, 8 * D), lambda i: (0, 0)),
            pl.BlockSpec((1, 1), lambda i: (0, 0)),
            pl.BlockSpec((A, D), lambda i: (0, 0)),
            pl.BlockSpec((A, 1), lambda i: (0, 0)),
        ],
        out_specs=[
            pl.BlockSpec((R, 1), lambda i: (i, 0)),
            pl.BlockSpec((A, R), lambda i: (0, i)),
            pl.BlockSpec((1, A, R // CHUNK), lambda i: (i, 0, 0)),
        ],
        out_shape=[
            jax.ShapeDtypeStruct((NPAD, 1), jnp.float32),
            jax.ShapeDtypeStruct((A, NPAD), jnp.float32),
            jax.ShapeDtypeStruct((NBLK, A, R // CHUNK), jnp.float32),
        ],
    )(feat_pad, w1t, b1_2d, w2t, b2_2d, raw_ano, ano_idx)


def _merge16(vals, idxs, x, xi):
    xs, xis = plsc.sort_key_val(x, xi, descending=True)
    rxs = lax.rev(xs, (0,))
    rxis = lax.rev(xis, (0,))
    take_a = (vals > rxs) | ((vals == rxs) & (idxs < rxis))
    mv = jnp.where(take_a, vals, rxs)
    mi = jnp.where(take_a, idxs, rxis)
    nv, ni = plsc.sort_key_val(mv, mi, descending=True)
    return nv, ni


def _topk_body(sims_hbm, m_hbm, feat_hbm, vals_out, idx_out, near_out,
               buf, mbuf, idref, vout, iout, rows_v, sem):
    wid = lax.axis_index("s") * NC + lax.axis_index("c")
    iota16 = lax.iota(jnp.int32, L)
    minf = jnp.full((L,), NEG, dtype=jnp.float32)

    for rr in range(ROWS_PW):
        row = wid * ROWS_PW + rr
        pltpu.sync_copy(m_hbm.at[row], mbuf)

        def p1_body(g, carry):
            vals, thr = carry
            cm = mbuf[pl.ds(g * L, L)]

            def do_merge(c):
                rxs = lax.rev(plsc.sort_key_val(cm, iota16, descending=True)[0], (0,))
                nv = plsc.sort_key_val(jnp.maximum(c[0], rxs), iota16,
                                       descending=True)[0]
                return nv, jnp.min(nv)

            return lax.cond(jnp.any(cm > thr), do_merge, lambda c: c, carry)

        _, t0 = lax.fori_loop(0, NCH // L, p1_body, (minf, jnp.min(minf)))

        def pa_group(g, slot):
            cm = mbuf[pl.ds(g * L, L)]

            def hit_group(s):
                for lane in range(L):
                    def issue(s2, _lane=lane):
                        chunk = g * L + _lane
                        idref[s2] = chunk
                        pltpu.make_async_copy(
                            sims_hbm.at[row, pl.ds(chunk * CHUNK, CHUNK)],
                            buf.at[pl.ds(s2 * CHUNK, CHUNK)], sem).start()
                        return s2 + 1

                    s = lax.cond(cm[lane] >= t0, issue, lambda s2: s2, s)
                return s

            return lax.cond(jnp.any(cm >= t0), hit_group, lambda s: s, slot)

        slot = lax.fori_loop(0, NCH // L, pa_group, jnp.int32(0))

        def pb(j, z):
            pltpu.make_async_copy(sims_hbm.at[row, pl.ds(0, CHUNK)],
                                  buf.at[pl.ds(0, CHUNK)], sem).wait()
            return z

        lax.fori_loop(0, slot, pb, jnp.int32(0))

        def pc(j, carry):
            chunk = idref[j]
            cb = j * CHUNK
            for v in range(CHUNK // L):
                off = cb + v * L
                x = buf[pl.ds(off, L)]
                xi = chunk * CHUNK + v * L + iota16

                def mg(c, x=x, xi=xi):
                    nv, ni = _merge16(c[0], c[1], x, xi)
                    return nv, ni, jnp.min(nv)

                carry = lax.cond(jnp.any(x > carry[2]), mg,
                                 lambda c: c, carry)
            return carry

        vals, idxs, _ = lax.fori_loop(
            0, slot, pc,
            (minf, jnp.zeros((L,), jnp.int32), jnp.min(minf)))
        vout[...] = vals
        iout[...] = idxs
        pltpu.sync_copy(vout, vals_out.at[row])
        pltpu.sync_copy(iout, idx_out.at[row])
        ncp = pltpu.make_async_copy(feat_hbm.at[iout], rows_v, sem)
        ncp.start()
        ncp.wait()
        pltpu.sync_copy(rows_v, near_out.at[pl.ds(row * HIDDEN_K, HIDDEN_K)])


def _topk_pass(sims, m_arr, feat_emb):
    return pl.kernel(
        _topk_body,
        out_type=[
            jax.ShapeDtypeStruct((A, HIDDEN_K), jnp.float32),
            jax.ShapeDtypeStruct((A, HIDDEN_K), jnp.int32),
            jax.ShapeDtypeStruct((A * HIDDEN_K, D), jnp.float32),
        ],
        compiler_params=pltpu.CompilerParams(needs_layout_passes=False),
        mesh=plsc.VectorSubcoreMesh(**_SC_MESH),
        scratch_types=[
            pltpu.VMEM((NPAD,), jnp.float32),
            pltpu.VMEM((NCH,), jnp.float32),
            pltpu.SMEM((NCH,), jnp.int32),
            pltpu.VMEM((HIDDEN_K,), jnp.float32),
            pltpu.VMEM((HIDDEN_K,), jnp.int32),
            pltpu.VMEM((HIDDEN_K, D), jnp.float32),
            pltpu.SemaphoreType.DMA,
        ],
    )(sims, m_arr, feat_emb)


def _stats_body(raw_ano, raw_no, raw_near, std_ab_o, std_no_o, nei_ab_o, nei_no_o):
    a = raw_ano[...]
    an = a / jnp.maximum(jnp.sqrt(jnp.sum(a * a, axis=1, keepdims=True)), EPS)
    no = raw_no[...]
    non = no / jnp.maximum(jnp.sqrt(jnp.sum(no * no, axis=1, keepdims=True)), EPS)
    nr = raw_near[...]
    nrn = nr / jnp.maximum(jnp.sqrt(jnp.sum(nr * nr, axis=1, keepdims=True)), EPS)

    dn = (((1,), (1,)), ((), ()))
    std_ab_o[...] = jnp.mean(
        lax.dot_general(an, an, dn, preferred_element_type=jnp.float32),
        axis=1, keepdims=True)
    std_no_o[...] = jnp.mean(
        lax.dot_general(an, non, dn, preferred_element_type=jnp.float32),
        axis=1, keepdims=True)
    nei_ab_o[...] = jnp.mean(
        lax.dot_general(nrn, an, dn, preferred_element_type=jnp.float32),
        axis=1, keepdims=True)
    nei_no_o[...] = jnp.mean(
        lax.dot_general(nrn, non, dn, preferred_element_type=jnp.float32),
        axis=1, keepdims=True)


def _stats_pass(raw_ano, raw_no, raw_near):
    return pl.pallas_call(
        _stats_body,
        out_shape=[
            jax.ShapeDtypeStruct((A, 1), jnp.float32),
            jax.ShapeDtypeStruct((A, 1), jnp.float32),
            jax.ShapeDtypeStruct((A * HIDDEN_K, 1), jnp.float32),
            jax.ShapeDtypeStruct((A * HIDDEN_K, 1), jnp.float32),
        ],
    )(raw_ano, raw_no, raw_near)


def _rl_body(sims_near, std_ab, std_no, nei_ab, nei_no,
             rewards_o, newth_o, mask_o):
    sims = sims_near[...]
    sab = std_ab[...]
    sno = std_no[...]
    nab = nei_ab[...]
    nno = nei_no[...]
    th = jnp.full((A, 1), INI_TH, dtype=jnp.float32)
    rws = []
    for _ in range(EPISODES):
        cond = sims >= th
        good = jnp.where(cond, (nab >= sab).astype(jnp.float32),
                         (nno <= sno).astype(jnp.float32))
        reward = jnp.sum(good, axis=1, keepdims=True)
        th = jnp.where(reward >= 0.5 * HIDDEN_K, th - STEP_SIZE, th + STEP_SIZE)
        rws.append(reward)
    rewards_o[...] = jnp.concatenate(rws, axis=1)
    new_th = jnp.where(th >= 1.0, 0.999, th)
    new_th = jnp.where(new_th <= 0.0, 0.001, new_th)
    newth_o[...] = new_th
    mask_o[...] = (sims > new_th).astype(jnp.int32)


def _rl_pass(sims_near, std_ab, std_no, nei_ab, nei_no):
    return pl.pallas_call(
        _rl_body,
        out_shape=[
            jax.ShapeDtypeStruct((A, EPISODES), jnp.float32),
            jax.ShapeDtypeStruct((A, 1), jnp.float32),
            jax.ShapeDtypeStruct((A, HIDDEN_K), jnp.int32),
        ],
    )(sims_near, std_ab, std_no, nei_ab, nei_no)


def kernel(feat_emb, nodes_batch, labels_batch, anomaly_list, norm_list, W1, b1, W2, b2):
    feat_pad = jnp.pad(feat_emb, ((0, NPAD - N), (0, 0)))
    w1t = W1.T
    b1_2d = b1.reshape(1, 8 * D)
    b2_2d = b2.reshape(1, 1)

    idx_pad = jnp.concatenate([
        anomaly_list.astype(jnp.int32), norm_list.astype(jnp.int32),
        jnp.zeros((GB - A - NNORM,), jnp.int32)])
    gath = _gather_rows(feat_emb, idx_pad)
    raw_ano = gath[:A]
    raw_no = gath[A:A + NNORM]

    ano_idx = anomaly_list.astype(jnp.int32).reshape(A, 1)
    pred_pad, sims, m3 = _main_pass(feat_pad, w1t, b1_2d, W2, b2_2d,
                                    raw_ano, ano_idx)
    pred_score = pred_pad[:N]
    m_arr = m3.transpose(1, 0, 2).reshape(A, NCH)

    _TOPK_SC = True
    if _TOPK_SC:
        sims_near, nearest, raw_near = _topk_pass(sims, m_arr, feat_emb)
    else:
        sims_near, nearest = lax.top_k(sims, HIDDEN_K)
        raw_near = feat_emb[nearest.reshape(-1)]

    std_ab, std_no, nei_ab_f, nei_no_f = _stats_pass(raw_ano, raw_no, raw_near)
    rewards, newth, mask_i = _rl_pass(
        sims_near, std_ab, std_no,
        nei_ab_f.reshape(A, HIDDEN_K), nei_no_f.reshape(A, HIDDEN_K))

    return (nearest, sims_near, mask_i.astype(jnp.bool_), rewards,
            newth.reshape(A), pred_score)
